# baseline (speedup 1.0000x reference)
import jax
import jax.numpy as jnp
from jax import lax
from jax.experimental import pallas as pl
from jax.experimental.pallas import tpu as pltpu
from jax.experimental.pallas import tpu_sc as plsc

B = 4
T = 2048
H = 512
N = 256
NSPAN = B * N
OUT_D = 4 * H

NC = 2
NS = 16
NW = NC * NS
WPB = N // 32
SPW = NSPAN // NW
CH = 8
NCHUNK = SPW // CH
NPAIR = NCHUNK // 2
L = 16
G = H // L


def _body(x_hbm, ij_hbm, out_hbm, ij_v, idx_v, rows_v, out_v,
          sem_g0, sem_g1, sem_w0, sem_w1):
  sem_g = (sem_g0, sem_g1)
  sem_w = (sem_w0, sem_w1)
  wid = lax.axis_index("s") * NC + lax.axis_index("c")
  base = wid * SPW
  b_idx = wid // WPB
  row_base = b_idx * T

  cp_i = pltpu.async_copy(ij_hbm.at[0, pl.ds(base, SPW)], ij_v.at[0], sem_g0)
  cp_j = pltpu.async_copy(ij_hbm.at[1, pl.ds(base, SPW)], ij_v.at[1], sem_g0)
  cp_i.wait()
  cp_j.wait()

  zeros16 = jnp.zeros((L,), jnp.int32)
  ones16 = jnp.full((L,), 1, jnp.int32)

  def pair_ij(p):
    i16 = ij_v[0, pl.ds(p * L, L)]
    j16 = ij_v[1, pl.ds(p * L, L)]
    return i16, j16

  def prep(p):
    pb = (p % 2) * 64
    i16, j16 = pair_ij(p)
    idx_v[pl.ds(pb + 0 * L, L)] = j16 + row_base
    idx_v[pl.ds(pb + 1 * L, L)] = i16 + row_base
    idx_v[pl.ds(pb + 2 * L, L)] = jnp.maximum(i16 - 1, 0) + row_base
    idx_v[pl.ds(pb + 3 * L, L)] = jnp.minimum(j16 + 1, T - 1) + row_base

  def fire(p, half, rb):
    pb = (p % 2) * 64
    off = half * CH
    pltpu.async_copy(
        x_hbm.at[idx_v.at[pl.ds(pb + 0 * L + off, CH)], pl.ds(0, H)],
        rows_v.at[rb, pl.ds(0, CH)], sem_g[rb])
    pltpu.async_copy(
        x_hbm.at[idx_v.at[pl.ds(pb + 1 * L + off, CH)], pl.ds(H, H)],
        rows_v.at[rb, pl.ds(CH, CH)], sem_g[rb])
    pltpu.async_copy(
        x_hbm.at[idx_v.at[pl.ds(pb + 2 * L + off, CH)], pl.ds(0, H)],
        out_v.at[rb, :, pl.ds(2 * H, H)], sem_g[rb])
    pltpu.async_copy(
        x_hbm.at[idx_v.at[pl.ds(pb + 3 * L + off, CH)], pl.ds(H, H)],
        out_v.at[rb, :, pl.ds(3 * H, H)], sem_g[rb])

  def wait_gathers(rb):
    pltpu.make_async_copy(x_hbm.at[pl.ds(0, CH), pl.ds(0, H)],
                          rows_v.at[rb, pl.ds(0, CH)], sem_g[rb]).wait()
    pltpu.make_async_copy(x_hbm.at[pl.ds(0, CH), pl.ds(0, H)],
                          rows_v.at[rb, pl.ds(CH, CH)], sem_g[rb]).wait()
    pltpu.make_async_copy(x_hbm.at[pl.ds(0, CH), pl.ds(0, H)],
                          out_v.at[rb, :, pl.ds(2 * H, H)], sem_g[rb]).wait()
    pltpu.make_async_copy(x_hbm.at[pl.ds(0, CH), pl.ds(0, H)],
                          out_v.at[rb, :, pl.ds(3 * H, H)], sem_g[rb]).wait()

  def wait_write(rb):
    pltpu.make_async_copy(out_v.at[rb], out_hbm.at[pl.ds(0, CH)],
                          sem_w[rb]).wait()

  def fixup(p, half, rb):
    i16, j16 = pair_ij(p)
    need = jnp.where((i16 == 0) | (j16 >= T - 1), ones16, zeros16)
    any_need = lax.reduce_max(need, (0,))

    @pl.when(any_need > 0)
    def _():
      one = jnp.full((L,), 1.0, jnp.float32)
      zero = jnp.zeros((L,), jnp.float32)
      k16 = jnp.where((i16 != 0) | (j16 != 0), one, zero)
      a16 = jnp.where(i16 >= 1, k16, zero)
      c16 = jnp.where(j16 < T - 1, k16, zero)

      def span_fix(s, _):
        sidx = jnp.full((L,), half * CH, jnp.int32) + s
        kk = k16.at[sidx].get(mode="promise_in_bounds")
        aa = a16.at[sidx].get(mode="promise_in_bounds")
        cc = c16.at[sidx].get(mode="promise_in_bounds")

        def fx(g, _):
          off = g * L
          rows_v[rb, s, pl.ds(off, L)] = rows_v[rb, s, pl.ds(off, L)] * kk
          rows_v[rb, CH + s, pl.ds(off, L)] = (
              rows_v[rb, CH + s, pl.ds(off, L)] * kk)
          out_v[rb, s, pl.ds(2 * H + off, L)] = (
              out_v[rb, s, pl.ds(2 * H + off, L)] * aa)
          out_v[rb, s, pl.ds(3 * H + off, L)] = (
              out_v[rb, s, pl.ds(3 * H + off, L)] * cc)
          return 0

        lax.fori_loop(0, G, fx, 0, unroll=2)
        return 0

      lax.fori_loop(0, CH, span_fix, 0)

  def compute_half(rb, lo):
    @plsc.parallel_loop(0, CH // 2, 1)
    def span_body(s0):
      s = lo + s0

      @plsc.parallel_loop(0, G, 1, unroll=8)
      def grp_body(g):
        off = g * L
        fend = rows_v[rb, s, pl.ds(off, L)]
        bsta = rows_v[rb, CH + s, pl.ds(off, L)]
        fpre = out_v[rb, s, pl.ds(2 * H + off, L)]
        bpos = out_v[rb, s, pl.ds(3 * H + off, L)]
        out_v[rb, s, pl.ds(off, L)] = fend - fpre
        out_v[rb, s, pl.ds(H + off, L)] = bsta - bpos

  def compute(c, rb):
    compute_half(rb, 0)
    pltpu.async_copy(out_v.at[rb, pl.ds(0, CH // 2)],
                     out_hbm.at[pl.ds(base + c * CH, CH // 2)], sem_w[rb])
    compute_half(rb, CH // 2)
    pltpu.async_copy(
        out_v.at[rb, pl.ds(CH // 2, CH // 2)],
        out_hbm.at[pl.ds(base + c * CH + CH // 2, CH // 2)], sem_w[rb])

  prep(0)
  fire(0, 0, 0)

  def pair_body(p, _):
    @pl.when(p + 1 < NPAIR)
    def _():
      prep(p + 1)

    fire(p, 1, 1)

    @pl.when(p >= 1)
    def _():
      wait_write(0)

    wait_gathers(0)
    fixup(p, 0, 0)
    compute(2 * p, 0)

    @pl.when(p + 1 < NPAIR)
    def _():
      fire(p + 1, 0, 0)

    @pl.when(p >= 1)
    def _():
      wait_write(1)

    wait_gathers(1)
    fixup(p, 1, 1)
    compute(2 * p + 1, 1)
    return 0

  lax.fori_loop(0, NPAIR, pair_body, 0)
  wait_write(0)
  wait_write(1)


@jax.jit
def _launch(x2, span_idxs):
  mesh = plsc.VectorSubcoreMesh(core_axis_name="c", subcore_axis_name="s")
  return pl.kernel(
      _body,
      out_type=jax.ShapeDtypeStruct((NSPAN, OUT_D), jnp.float32),
      mesh=mesh,
      compiler_params=pltpu.CompilerParams(needs_layout_passes=False),
      scratch_types=[
          pltpu.VMEM((2, SPW), jnp.int32),
          pltpu.VMEM((128,), jnp.int32),
          pltpu.VMEM((2, 2 * CH, H), jnp.float32),
          pltpu.VMEM((2, CH, OUT_D), jnp.float32),
          pltpu.SemaphoreType.DMA,
          pltpu.SemaphoreType.DMA,
          pltpu.SemaphoreType.DMA,
          pltpu.SemaphoreType.DMA,
      ],
  )(x2, span_idxs)


def kernel(input, span_idxs):
  x2 = input.reshape(B * T, 2 * H)
  ij = span_idxs.reshape(NSPAN, 2).astype(jnp.int32)
  ij2 = jnp.stack([ij[:, 0], ij[:, 1]])
  out = _launch(x2, ij2)
  return out.reshape(B, N, OUT_D)


# speedup vs baseline: 1.0108x; 1.0108x over previous
import jax
import jax.numpy as jnp
from jax import lax
from jax.experimental import pallas as pl
from jax.experimental.pallas import tpu as pltpu
from jax.experimental.pallas import tpu_sc as plsc

B = 4
T = 2048
H = 512
N = 256
NSPAN = B * N
OUT_D = 4 * H

NC = 2
NS = 16
NW = NC * NS
WPB = N // 32
SPW = NSPAN // NW
CH = 8
NCHUNK = SPW // CH
NPAIR = NCHUNK // 2
L = 16
G = H // L


def _body(x_hbm, ij_hbm, out_hbm, ij_v, idx_v, rows_v, out_v,
          sem_g0, sem_g1, sem_w0, sem_w1):
  sem_g = (sem_g0, sem_g1)
  sem_w = (sem_w0, sem_w1)
  wid = lax.axis_index("s") * NC + lax.axis_index("c")
  base = wid * SPW
  b_idx = wid // WPB
  row_base = b_idx * T

  cp_i = pltpu.async_copy(ij_hbm.at[0, pl.ds(base, SPW)], ij_v.at[0], sem_g0)
  cp_j = pltpu.async_copy(ij_hbm.at[1, pl.ds(base, SPW)], ij_v.at[1], sem_g0)
  cp_i.wait()
  cp_j.wait()

  zeros16 = jnp.zeros((L,), jnp.int32)
  ones16 = jnp.full((L,), 1, jnp.int32)

  def pair_ij(p):
    i16 = ij_v[0, pl.ds(p * L, L)]
    j16 = ij_v[1, pl.ds(p * L, L)]
    return i16, j16

  def prep(p):
    pb = (p % 2) * 64
    i16, j16 = pair_ij(p)
    idx_v[pl.ds(pb + 0 * L, L)] = j16 + row_base
    idx_v[pl.ds(pb + 1 * L, L)] = i16 + row_base
    idx_v[pl.ds(pb + 2 * L, L)] = jnp.maximum(i16 - 1, 0) + row_base
    idx_v[pl.ds(pb + 3 * L, L)] = jnp.minimum(j16 + 1, T - 1) + row_base

  def fire(p, half, rb):
    pb = (p % 2) * 64
    off = half * CH
    pltpu.async_copy(
        x_hbm.at[idx_v.at[pl.ds(pb + 0 * L + off, CH)], pl.ds(0, H)],
        rows_v.at[rb, pl.ds(0, CH)], sem_g[rb])
    pltpu.async_copy(
        x_hbm.at[idx_v.at[pl.ds(pb + 1 * L + off, CH)], pl.ds(H, H)],
        rows_v.at[rb, pl.ds(CH, CH)], sem_g[rb])
    pltpu.async_copy(
        x_hbm.at[idx_v.at[pl.ds(pb + 2 * L + off, CH)], pl.ds(0, H)],
        out_v.at[rb, :, pl.ds(2 * H, H)], sem_g[rb])
    pltpu.async_copy(
        x_hbm.at[idx_v.at[pl.ds(pb + 3 * L + off, CH)], pl.ds(H, H)],
        out_v.at[rb, :, pl.ds(3 * H, H)], sem_g[rb])

  def wait_gathers(rb):
    pltpu.make_async_copy(x_hbm.at[pl.ds(0, CH), pl.ds(0, H)],
                          rows_v.at[rb, pl.ds(0, CH)], sem_g[rb]).wait()
    pltpu.make_async_copy(x_hbm.at[pl.ds(0, CH), pl.ds(0, H)],
                          rows_v.at[rb, pl.ds(CH, CH)], sem_g[rb]).wait()
    pltpu.make_async_copy(x_hbm.at[pl.ds(0, CH), pl.ds(0, H)],
                          out_v.at[rb, :, pl.ds(2 * H, H)], sem_g[rb]).wait()
    pltpu.make_async_copy(x_hbm.at[pl.ds(0, CH), pl.ds(0, H)],
                          out_v.at[rb, :, pl.ds(3 * H, H)], sem_g[rb]).wait()

  def wait_write(rb):
    pltpu.make_async_copy(out_v.at[rb], out_hbm.at[pl.ds(0, CH)],
                          sem_w[rb]).wait()

  def fixup(p, half, rb):
    i16, j16 = pair_ij(p)
    need = jnp.where((i16 == 0) | (j16 >= T - 1), ones16, zeros16)
    any_need = lax.reduce_max(need, (0,))

    @pl.when(any_need > 0)
    def _():
      one = jnp.full((L,), 1.0, jnp.float32)
      zero = jnp.zeros((L,), jnp.float32)
      k16 = jnp.where((i16 != 0) | (j16 != 0), one, zero)
      a16 = jnp.where(i16 >= 1, k16, zero)
      c16 = jnp.where(j16 < T - 1, k16, zero)

      def span_fix(s, _):
        sidx = jnp.full((L,), half * CH, jnp.int32) + s
        kk = k16.at[sidx].get(mode="promise_in_bounds")
        aa = a16.at[sidx].get(mode="promise_in_bounds")
        cc = c16.at[sidx].get(mode="promise_in_bounds")

        def fx(g, _):
          off = g * L
          rows_v[rb, s, pl.ds(off, L)] = rows_v[rb, s, pl.ds(off, L)] * kk
          rows_v[rb, CH + s, pl.ds(off, L)] = (
              rows_v[rb, CH + s, pl.ds(off, L)] * kk)
          out_v[rb, s, pl.ds(2 * H + off, L)] = (
              out_v[rb, s, pl.ds(2 * H + off, L)] * aa)
          out_v[rb, s, pl.ds(3 * H + off, L)] = (
              out_v[rb, s, pl.ds(3 * H + off, L)] * cc)
          return 0

        lax.fori_loop(0, G, fx, 0, unroll=2)
        return 0

      lax.fori_loop(0, CH, span_fix, 0)

  def compute(rb):
    @plsc.parallel_loop(0, CH, 1)
    def span_body(s):
      @plsc.parallel_loop(0, G, 1, unroll=8)
      def grp_body(g):
        off = g * L
        fend = rows_v[rb, s, pl.ds(off, L)]
        bsta = rows_v[rb, CH + s, pl.ds(off, L)]
        fpre = out_v[rb, s, pl.ds(2 * H + off, L)]
        bpos = out_v[rb, s, pl.ds(3 * H + off, L)]
        out_v[rb, s, pl.ds(off, L)] = fend - fpre
        out_v[rb, s, pl.ds(H + off, L)] = bsta - bpos

  def write(c, rb):
    pltpu.async_copy(out_v.at[rb],
                     out_hbm.at[pl.ds(base + c * CH, CH)], sem_w[rb])

  prep(0)
  fire(0, 0, 0)

  def pair_body(p, _):
    @pl.when(p + 1 < NPAIR)
    def _():
      prep(p + 1)

    fire(p, 1, 1)

    @pl.when(p >= 1)
    def _():
      wait_write(0)

    wait_gathers(0)
    fixup(p, 0, 0)
    compute(0)
    write(2 * p, 0)

    @pl.when(p + 1 < NPAIR)
    def _():
      fire(p + 1, 0, 0)

    @pl.when(p >= 1)
    def _():
      wait_write(1)

    wait_gathers(1)
    fixup(p, 1, 1)
    compute(1)
    write(2 * p + 1, 1)
    return 0

  lax.fori_loop(0, NPAIR, pair_body, 0)
  wait_write(0)
  wait_write(1)


@jax.jit
def _launch(x2, span_idxs):
  mesh = plsc.VectorSubcoreMesh(core_axis_name="c", subcore_axis_name="s")
  return pl.kernel(
      _body,
      out_type=jax.ShapeDtypeStruct((NSPAN, OUT_D), jnp.float32),
      mesh=mesh,
      compiler_params=pltpu.CompilerParams(needs_layout_passes=False),
      scratch_types=[
          pltpu.VMEM((2, SPW), jnp.int32),
          pltpu.VMEM((128,), jnp.int32),
          pltpu.VMEM((2, 2 * CH, H), jnp.float32),
          pltpu.VMEM((2, CH, OUT_D), jnp.float32),
          pltpu.SemaphoreType.DMA,
          pltpu.SemaphoreType.DMA,
          pltpu.SemaphoreType.DMA,
          pltpu.SemaphoreType.DMA,
      ],
  )(x2, span_idxs)


def kernel(input, span_idxs):
  x2 = input.reshape(B * T, 2 * H)
  ij = span_idxs.reshape(NSPAN, 2).astype(jnp.int32)
  ij2 = jnp.stack([ij[:, 0], ij[:, 1]])
  out = _launch(x2, ij2)
  return out.reshape(B, N, OUT_D)
